# baseline (speedup 1.0000x reference)
import jax
import jax.numpy as jnp
from jax.experimental import pallas as pl
from jax.experimental.pallas import tpu as pltpu

N = 1024
INV_DEG = 1.0 / (N + 1)
EPS = 1e-5


def _layer_norm_scaled(u, g, b):
    m = jnp.mean(u, axis=-1, keepdims=True)
    d = u - m
    v = jnp.mean(d * d, axis=-1, keepdims=True)
    return d * jax.lax.rsqrt(v + EPS * (N + 1.0) ** 2) * g + b


def _matmul_t(a, w):
    return jax.lax.dot_general(a, w, (((1,), (1,)), ((), ())),
                               preferred_element_type=jnp.float32)


def _fused_kernel(x_ref, w1t_ref, b1_ref, w2t_ref, b2_ref,
                  g1_ref, be1_ref, g2_ref, be2_ref,
                  f1t_ref, f1b_ref, f2t_ref, f2b_ref, f3t_ref, f3b_ref,
                  out_ref):
    x = x_ref[:]

    hw1 = _matmul_t(x, w1t_ref[:])
    s1 = jnp.sum(hw1, axis=0, keepdims=True) + b1_ref[:] * (N + 1.0)
    u1 = jnp.maximum(hw1 + s1, 0.0)
    h = _layer_norm_scaled(u1, g1_ref[:], be1_ref[:])

    hw2 = _matmul_t(h, w2t_ref[:])
    s2 = jnp.sum(hw2, axis=0, keepdims=True) + b2_ref[:] * (N + 1.0)
    g = _layer_norm_scaled(hw2 + s2, g2_ref[:] * 0.5, be2_ref[:] * 0.5)

    f = jnp.maximum(_matmul_t(x, f1t_ref[:]) + f1b_ref[:], 0.0)
    f = jnp.maximum(_matmul_t(f, f2t_ref[:]) + f2b_ref[:], 0.0)
    f = _matmul_t(f, f3t_ref[:] * 0.5) + f3b_ref[:] * 0.5

    out_ref[:] = g + f


def kernel(x, edge_index, conv1_w, conv1_b, conv2_w, conv2_b,
           norm1_g, norm1_b, norm2_g, norm2_b,
           fc1_w, fc1_b, fc2_w, fc2_b, fc3_w, fc3_b):
    del edge_index
    row = lambda v: v.reshape(1, -1)
    operands = (
        x,
        conv1_w, row(conv1_b),
        conv2_w, row(conv2_b),
        row(norm1_g), row(norm1_b), row(norm2_g), row(norm2_b),
        fc1_w, row(fc1_b),
        fc2_w, row(fc2_b),
        fc3_w, row(fc3_b),
    )
    return pl.pallas_call(
        _fused_kernel,
        out_shape=jax.ShapeDtypeStruct(x.shape, jnp.float32),
        in_specs=[pl.BlockSpec(memory_space=pltpu.MemorySpace.VMEM)
                  for _ in operands],
        out_specs=pl.BlockSpec(memory_space=pltpu.MemorySpace.VMEM),
    )(*operands)


# speedup vs baseline: 1.0153x; 1.0153x over previous
import jax
import jax.numpy as jnp
from jax.experimental import pallas as pl
from jax.experimental.pallas import tpu as pltpu

N = 1024
INV_DEG = 1.0 / (N + 1)
EPS = 1e-5


def _layer_norm_scaled(u, g, b):
    m = jnp.mean(u, axis=-1, keepdims=True)
    d = u - m
    v = jnp.mean(d * d, axis=-1, keepdims=True)
    return d * jax.lax.rsqrt(v + EPS * (N + 1.0) ** 2) * g + b


def _matmul_t(a, w):
    return jax.lax.dot_general(a, w, (((1,), (1,)), ((), ())),
                               preferred_element_type=jnp.float32)


def _fused_kernel(x_ref, w1t_ref, b1_ref, w2t_ref, b2_ref,
                  g1_ref, be1_ref, g2_ref, be2_ref,
                  f1t_ref, f1b_ref, f2t_ref, f2b_ref, f3t_ref, f3b_ref,
                  out_ref):
    x = x_ref[:]

    hw1 = _matmul_t(x, w1t_ref[:])
    f = jnp.maximum(_matmul_t(x, f1t_ref[:]) + f1b_ref[:], 0.0)
    s1 = jnp.sum(hw1, axis=0, keepdims=True) + b1_ref[:] * (N + 1.0)
    u1 = jnp.maximum(hw1 + s1, 0.0)
    h = _layer_norm_scaled(u1, g1_ref[:], be1_ref[:])

    f = jnp.maximum(_matmul_t(f, f2t_ref[:]) + f2b_ref[:], 0.0)

    hw2 = _matmul_t(h, w2t_ref[:])
    f = _matmul_t(f, f3t_ref[:] * 0.5) + f3b_ref[:] * 0.5
    s2 = jnp.sum(hw2, axis=0, keepdims=True) + b2_ref[:] * (N + 1.0)
    g = _layer_norm_scaled(hw2 + s2, g2_ref[:] * 0.5, be2_ref[:] * 0.5)

    out_ref[:] = g + f


def kernel(x, edge_index, conv1_w, conv1_b, conv2_w, conv2_b,
           norm1_g, norm1_b, norm2_g, norm2_b,
           fc1_w, fc1_b, fc2_w, fc2_b, fc3_w, fc3_b):
    del edge_index
    row = lambda v: v.reshape(1, -1)
    operands = (
        x,
        conv1_w, row(conv1_b),
        conv2_w, row(conv2_b),
        row(norm1_g), row(norm1_b), row(norm2_g), row(norm2_b),
        fc1_w, row(fc1_b),
        fc2_w, row(fc2_b),
        fc3_w, row(fc3_b),
    )
    return pl.pallas_call(
        _fused_kernel,
        out_shape=jax.ShapeDtypeStruct(x.shape, jnp.float32),
        in_specs=[pl.BlockSpec(memory_space=pltpu.MemorySpace.VMEM)
                  for _ in operands],
        out_specs=pl.BlockSpec(memory_space=pltpu.MemorySpace.VMEM),
    )(*operands)
